# baseline (speedup 1.0000x reference)
import functools

import jax
import jax.numpy as jnp
from jax import lax
from jax.experimental import pallas as pl
from jax.experimental.pallas import tpu as pltpu
from jax.experimental.pallas import tpu_sc as plsc

_B = 4096
_L = 200
_D = 64
_EPS = 1e-12

_NR = 2 * _B
_CH = 112
_NCH = 2
_LP = _CH * _NCH
_NW = 32
_RW = _NR // _NW
_LANES = 16
_KD = _D // _LANES
_UNROLL = 8


def _sc_pool_body(idx_hbm, table_hbm, out_hbm, idxbuf, buf, outbuf, t0v,
                  sem0, sem1):
    wid = lax.axis_index("s") * 2 + lax.axis_index("c")
    base = wid * _RW
    sems = (sem0, sem1)

    pltpu.sync_copy(idx_hbm.at[pl.ds(base, _RW)], idxbuf)
    pltpu.sync_copy(table_hbm.at[0], t0v)
    t0 = [t0v[pl.ds(k * _LANES, _LANES)] for k in range(_KD)]

    def gather_descs(slot, local):
        return [
            pltpu.make_async_copy(
                table_hbm.at[idxbuf.at[local, j]],
                buf.at[slot, pl.ds(j * _CH, _CH)],
                sems[slot],
            )
            for j in range(_NCH)
        ]

    def fire(slot, local):
        for dsc in gather_descs(slot, local):
            dsc.start()

    def consume(slot, local_row):
        for dsc in gather_descs(slot, local_row):
            dsc.wait()
        one = jnp.ones((_LANES,), jnp.float32)
        zv = jnp.zeros((_LANES,), jnp.float32)
        nzv = jnp.zeros((_LANES,), jnp.float32)
        for j in range(_NCH):
            for c in range(_CH // _LANES):
                v = idxbuf[local_row, j, pl.ds(c * _LANES, _LANES)]
                nzv = nzv + jnp.where(v == 0, one, zv)
        nzf = jnp.broadcast_to(jnp.sum(nzv), (_LANES,))
        cnt = jnp.float32(_LP) - nzf

        zero = jnp.zeros((_LANES,), jnp.float32)
        parts = [[zero] * 4 for _ in range(_KD)]
        for r in range(_LP):
            for k in range(_KD):
                parts[k][r % 4] = parts[k][r % 4] + buf[
                    slot, r, pl.ds(k * _LANES, _LANES)]
        inv = 1.0 / cnt
        for k in range(_KD):
            acc = (parts[k][0] + parts[k][1]) + (parts[k][2] + parts[k][3])
            outbuf[local_row, pl.ds(k * _LANES, _LANES)] = (
                (acc - nzf * t0[k]) * inv)

    fire(0, 0)

    def outer(i, carry):
        for phase in range(2):
            local = 2 * i + phase
            nxt = local + 1

            @pl.when(nxt < _RW)
            def _():
                fire(1 - phase, nxt)

            consume(phase, local)
        return carry

    lax.fori_loop(0, _RW // 2, outer, 0)
    pltpu.sync_copy(outbuf, out_hbm.at[pl.ds(base, _RW)])


_sc_pool = functools.partial(
    pl.kernel,
    mesh=plsc.VectorSubcoreMesh(core_axis_name="c", subcore_axis_name="s"),
    compiler_params=pltpu.CompilerParams(
        needs_layout_passes=False, use_tc_tiling_on_sc=False),
    out_type=jax.ShapeDtypeStruct((_NR, _D), jnp.float32),
    scratch_types=[
        pltpu.VMEM((_RW, _NCH, _CH), jnp.int32),
        pltpu.VMEM((2, _LP, _D), jnp.float32),
        pltpu.VMEM((_RW, _D), jnp.float32),
        pltpu.VMEM((_D,), jnp.float32),
        pltpu.SemaphoreType.DMA,
        pltpu.SemaphoreType.DMA,
    ],
)(_sc_pool_body)


def _ln_body(x_ref, g_ref, b_ref, o_ref):
    x = x_ref[...]
    mu = jnp.mean(x, axis=-1, keepdims=True)
    xc = x - mu
    var = jnp.mean(xc * xc, axis=-1, keepdims=True)
    o_ref[...] = xc * lax.rsqrt(var + _EPS) * g_ref[...] + b_ref[...]


def _layernorm(pooled, gamma, beta):
    blk = 1024
    return pl.pallas_call(
        _ln_body,
        grid=(_NR // blk,),
        in_specs=[
            pl.BlockSpec((blk, _D), lambda i: (i, 0)),
            pl.BlockSpec((1, _D), lambda i: (0, 0)),
            pl.BlockSpec((1, _D), lambda i: (0, 0)),
        ],
        out_specs=pl.BlockSpec((blk, _D), lambda i: (i, 0)),
        out_shape=jax.ShapeDtypeStruct((_NR, _D), jnp.float32),
    )(pooled, gamma, beta)


def kernel(x_s, x_t, table, gamma, beta):
    idx = jnp.concatenate(
        [x_s.astype(jnp.int32), x_t.astype(jnp.int32)], axis=0)
    idx = jnp.pad(idx, ((0, 0), (0, _LP - _L)))
    idx = idx.reshape(_NR, _NCH, _CH)
    pooled = _sc_pool(idx, table)
    out = _layernorm(pooled, gamma.reshape(1, _D), beta.reshape(1, _D))
    return out[:_B], out[_B:]


# speedup vs baseline: 2.1939x; 2.1939x over previous
import functools

import jax
import jax.numpy as jnp
from jax import lax
from jax.experimental import pallas as pl
from jax.experimental.pallas import tpu as pltpu
from jax.experimental.pallas import tpu_sc as plsc

_B = 4096
_L = 200
_D = 64
_EPS = 1e-12

_NR = 2 * _B
_LANES = 16
_NCH = 13
_LP = _NCH * _LANES
_NW = 32
_RW = _NR // _NW
_KD = _D // _LANES


def _sc_pool_body(idx_hbm, table_hbm, out_hbm, idxbuf, buf, outbuf, t0v,
                  sem0, sem1):
    wid = lax.axis_index("s") * 2 + lax.axis_index("c")
    base = wid * _RW
    sems = (sem0, sem1)

    pltpu.sync_copy(idx_hbm.at[pl.ds(base, _RW)], idxbuf)
    pltpu.sync_copy(table_hbm.at[0], t0v)
    t0 = [t0v[pl.ds(k * _LANES, _LANES)] for k in range(_KD)]

    def idx_chunks(local):
        return [idxbuf[local, pl.ds(c * _LANES, _LANES)]
                for c in range(_NCH)]

    def gather_descs(slot, local):
        return [
            pltpu.make_async_copy(
                table_hbm.at[iv],
                buf.at[slot, pl.ds(c * _LANES, _LANES)],
                sems[slot],
            )
            for c, iv in enumerate(idx_chunks(local))
        ]

    def fire(slot, local):
        for dsc in gather_descs(slot, local):
            dsc.start()

    def consume(slot, local_row):
        ivs = idx_chunks(local_row)
        for dsc in gather_descs(slot, local_row):
            dsc.wait()
        one = jnp.ones((_LANES,), jnp.float32)
        zv = jnp.zeros((_LANES,), jnp.float32)
        nzv = jnp.zeros((_LANES,), jnp.float32)
        for iv in ivs:
            nzv = nzv + jnp.where(iv == 0, one, zv)
        nzf = jnp.broadcast_to(jnp.sum(nzv), (_LANES,))
        cnt = jnp.float32(_LP) - nzf

        zero = jnp.zeros((_LANES,), jnp.float32)
        parts = [[zero] * 4 for _ in range(_KD)]
        for r in range(_LP):
            for k in range(_KD):
                parts[k][r % 4] = parts[k][r % 4] + buf[
                    slot, r, pl.ds(k * _LANES, _LANES)]
        inv = 1.0 / cnt
        for k in range(_KD):
            acc = (parts[k][0] + parts[k][1]) + (parts[k][2] + parts[k][3])
            outbuf[local_row, pl.ds(k * _LANES, _LANES)] = (
                (acc - nzf * t0[k]) * inv)

    fire(0, 0)

    def outer(i, carry):
        for phase in range(2):
            local = 2 * i + phase
            nxt = local + 1

            @pl.when(nxt < _RW)
            def _():
                fire(1 - phase, nxt)

            consume(phase, local)
        return carry

    lax.fori_loop(0, _RW // 2, outer, 0)
    pltpu.sync_copy(outbuf, out_hbm.at[pl.ds(base, _RW)])


_sc_pool = functools.partial(
    pl.kernel,
    mesh=plsc.VectorSubcoreMesh(core_axis_name="c", subcore_axis_name="s"),
    compiler_params=pltpu.CompilerParams(
        needs_layout_passes=False, use_tc_tiling_on_sc=False),
    out_type=jax.ShapeDtypeStruct((_NR, _D), jnp.float32),
    scratch_types=[
        pltpu.VMEM((_RW, _LP), jnp.int32),
        pltpu.VMEM((2, _LP, _D), jnp.float32),
        pltpu.VMEM((_RW, _D), jnp.float32),
        pltpu.VMEM((_D,), jnp.float32),
        pltpu.SemaphoreType.DMA,
        pltpu.SemaphoreType.DMA,
    ],
)(_sc_pool_body)


def _ln_body(x_ref, g_ref, b_ref, o_ref):
    x = x_ref[...]
    mu = jnp.mean(x, axis=-1, keepdims=True)
    xc = x - mu
    var = jnp.mean(xc * xc, axis=-1, keepdims=True)
    o_ref[...] = xc * lax.rsqrt(var + _EPS) * g_ref[...] + b_ref[...]


def _layernorm(pooled, gamma, beta):
    blk = 1024
    return pl.pallas_call(
        _ln_body,
        grid=(_NR // blk,),
        in_specs=[
            pl.BlockSpec((blk, _D), lambda i: (i, 0)),
            pl.BlockSpec((1, _D), lambda i: (0, 0)),
            pl.BlockSpec((1, _D), lambda i: (0, 0)),
        ],
        out_specs=pl.BlockSpec((blk, _D), lambda i: (i, 0)),
        out_shape=jax.ShapeDtypeStruct((_NR, _D), jnp.float32),
    )(pooled, gamma, beta)


def kernel(x_s, x_t, table, gamma, beta):
    idx = jnp.concatenate(
        [x_s.astype(jnp.int32), x_t.astype(jnp.int32)], axis=0)
    idx = jnp.pad(idx, ((0, 0), (0, _LP - _L)))
    pooled = _sc_pool(idx, table)
    out = _layernorm(pooled, gamma.reshape(1, _D), beta.reshape(1, _D))
    return out[:_B], out[_B:]


# speedup vs baseline: 2.1970x; 1.0014x over previous
import functools

import jax
import jax.numpy as jnp
from jax import lax
from jax.experimental import pallas as pl
from jax.experimental.pallas import tpu as pltpu
from jax.experimental.pallas import tpu_sc as plsc

_B = 4096
_L = 200
_D = 64
_EPS = 1e-12

_NR = 2 * _B
_LANES = 16
_NCH = 13
_LP = _NCH * _LANES
_NW = 32
_RW = _NR // _NW
_RH = _RW // 2
_KD = _D // _LANES
_NSLOT = 4


def _sc_pool_body(idx_hbm, table_hbm, out_hbm, idxbuf, buf, outbuf, t0v,
                  *sems):
    wid = lax.axis_index("s") * 2 + lax.axis_index("c")
    base = wid * _RW

    pltpu.sync_copy(table_hbm.at[0], t0v)
    t0 = [t0v[pl.ds(k * _LANES, _LANES)] for k in range(_KD)]

    def idx_chunks(local):
        return [idxbuf[local, pl.ds(c * _LANES, _LANES)]
                for c in range(_NCH)]

    def gather_descs(slot, local):
        return [
            pltpu.make_async_copy(
                table_hbm.at[iv],
                buf.at[slot, pl.ds(c * _LANES, _LANES)],
                sems[slot],
            )
            for c, iv in enumerate(idx_chunks(local))
        ]

    def fire(slot, local):
        for dsc in gather_descs(slot, local):
            dsc.start()

    def consume(slot, local_row):
        ivs = idx_chunks(local_row)
        for dsc in gather_descs(slot, local_row):
            dsc.wait()
        one = jnp.ones((_LANES,), jnp.float32)
        zv = jnp.zeros((_LANES,), jnp.float32)
        nzv = jnp.zeros((_LANES,), jnp.float32)
        for iv in ivs:
            nzv = nzv + jnp.where(iv == 0, one, zv)
        nzf = jnp.broadcast_to(jnp.sum(nzv), (_LANES,))
        cnt = jnp.float32(_LP) - nzf

        zero = jnp.zeros((_LANES,), jnp.float32)

        def acc_group(g, parts):
            parts = [list(p) for p in parts]
            rb = g * _LANES
            for u in range(_LANES):
                for k in range(_KD):
                    parts[k][u % 4] = parts[k][u % 4] + buf[
                        slot, rb + u, pl.ds(k * _LANES, _LANES)]
            return tuple(tuple(p) for p in parts)

        parts = lax.fori_loop(
            0, _NCH, acc_group, tuple((zero,) * 4 for _ in range(_KD)))
        inv = 1.0 / cnt
        for k in range(_KD):
            acc = (parts[k][0] + parts[k][1]) + (parts[k][2] + parts[k][3])
            outbuf[local_row, pl.ds(k * _LANES, _LANES)] = (
                (acc - nzf * t0[k]) * inv)

    def outer(i, carry):
        for phase in range(_NSLOT):
            local = _NSLOT * i + phase
            nxt = local + (_NSLOT - 1)

            @pl.when(nxt < _RH)
            def _():
                fire((phase + _NSLOT - 1) % _NSLOT, nxt)

            consume(phase, local)
        return carry

    for half in range(2):
        hbase = base + half * _RH
        pltpu.sync_copy(idx_hbm.at[pl.ds(hbase, _RH)], idxbuf)
        for p in range(_NSLOT - 1):
            fire(p, p)
        lax.fori_loop(0, _RH // _NSLOT, outer, 0)
        pltpu.sync_copy(outbuf, out_hbm.at[pl.ds(hbase, _RH)])


_sc_pool = functools.partial(
    pl.kernel,
    mesh=plsc.VectorSubcoreMesh(core_axis_name="c", subcore_axis_name="s"),
    compiler_params=pltpu.CompilerParams(
        needs_layout_passes=False, use_tc_tiling_on_sc=False),
    out_type=jax.ShapeDtypeStruct((_NR, _D), jnp.float32),
    scratch_types=[
        pltpu.VMEM((_RH, _LP), jnp.int32),
        pltpu.VMEM((_NSLOT, _LP, _D), jnp.float32),
        pltpu.VMEM((_RH, _D), jnp.float32),
        pltpu.VMEM((_D,), jnp.float32),
    ] + [pltpu.SemaphoreType.DMA] * _NSLOT,
)(_sc_pool_body)


def _ln_body(x_ref, g_ref, b_ref, o_ref):
    x = x_ref[...]
    mu = jnp.mean(x, axis=-1, keepdims=True)
    xc = x - mu
    var = jnp.mean(xc * xc, axis=-1, keepdims=True)
    o_ref[...] = xc * lax.rsqrt(var + _EPS) * g_ref[...] + b_ref[...]


def _layernorm(pooled, gamma, beta):
    blk = 1024
    return pl.pallas_call(
        _ln_body,
        grid=(_NR // blk,),
        in_specs=[
            pl.BlockSpec((blk, _D), lambda i: (i, 0)),
            pl.BlockSpec((1, _D), lambda i: (0, 0)),
            pl.BlockSpec((1, _D), lambda i: (0, 0)),
        ],
        out_specs=pl.BlockSpec((blk, _D), lambda i: (i, 0)),
        out_shape=jax.ShapeDtypeStruct((_NR, _D), jnp.float32),
    )(pooled, gamma, beta)


def kernel(x_s, x_t, table, gamma, beta):
    idx = jnp.concatenate(
        [x_s.astype(jnp.int32), x_t.astype(jnp.int32)], axis=0)
    idx = jnp.pad(idx, ((0, 0), (0, _LP - _L)))
    pooled = _sc_pool(idx, table)
    out = _layernorm(pooled, gamma.reshape(1, _D), beta.reshape(1, _D))
    return out[:_B], out[_B:]


# speedup vs baseline: 2.8584x; 1.3010x over previous
import functools

import jax
import jax.numpy as jnp
from jax import lax
from jax.experimental import pallas as pl
from jax.experimental.pallas import tpu as pltpu
from jax.experimental.pallas import tpu_sc as plsc

_B = 4096
_L = 200
_D = 64
_EPS = 1e-12

_NR = 2 * _B
_LANES = 16
_NCH = 13
_LP = _NCH * _LANES
_NW = 32
_RW = _NR // _NW
_RH = _RW // 2
_KD = _D // _LANES
_NSLOT = 4

def _sc_pool_body(idx_hbm, table_hbm, out_hbm, idxbuf, buf, outbuf, t0v,
                  *sems):
    wid = lax.axis_index("s") * 2 + lax.axis_index("c")
    base = wid * _RW

    pltpu.sync_copy(table_hbm.at[0], t0v)
    t0 = []
    for h in range(_KD // 2):
        a, b = plsc.unpack(t0v[pl.ds(2 * _LANES * h, 2 * _LANES)],
                           format=plsc.PackFormat.INTERLEAVED)
        t0 += [a, b]

    def idx_chunks(local):
        return [idxbuf[local, pl.ds(c * _LANES, _LANES)]
                for c in range(_NCH)]

    def gather_descs(slot, local):
        return [
            pltpu.make_async_copy(
                table_hbm.at[iv],
                buf.at[slot, pl.ds(c * _LANES, _LANES)],
                sems[slot],
            )
            for c, iv in enumerate(idx_chunks(local))
        ]

    def fire(slot, local):
        for dsc in gather_descs(slot, local):
            dsc.start()

    def consume(slot, local_row):
        ivs = idx_chunks(local_row)
        for dsc in gather_descs(slot, local_row):
            dsc.wait()
        one = jnp.ones((_LANES,), jnp.float32)
        zv = jnp.zeros((_LANES,), jnp.float32)
        nzv = jnp.zeros((_LANES,), jnp.float32)
        for iv in ivs:
            nzv = nzv + jnp.where(iv == 0, one, zv)
        nzf = jnp.broadcast_to(jnp.sum(nzv), (_LANES,))
        cnt = jnp.float32(_LP) - nzf

        zero = jnp.zeros((_LANES,), jnp.float32)

        def acc_group(g, parts):
            parts = [list(p) for p in parts]
            rb = g * _LANES
            for u in range(_LANES):
                for h in range(_KD // 2):
                    pk = buf[slot, rb + u,
                             pl.ds(2 * _LANES * h, 2 * _LANES)]
                    a, b = plsc.unpack(
                        pk, format=plsc.PackFormat.INTERLEAVED)
                    parts[2 * h][u % 4] = parts[2 * h][u % 4] + a
                    parts[2 * h + 1][u % 4] = parts[2 * h + 1][u % 4] + b
            return tuple(tuple(p) for p in parts)

        parts = lax.fori_loop(
            0, _NCH, acc_group, tuple((zero,) * 4 for _ in range(_KD)))
        inv = 1.0 / cnt
        for k in range(_KD):
            acc = (parts[k][0] + parts[k][1]) + (parts[k][2] + parts[k][3])
            outbuf[local_row, pl.ds(k * _LANES, _LANES)] = (
                (acc - nzf * t0[k]) * inv)

    def outer(i, carry):
        for phase in range(_NSLOT):
            local = _NSLOT * i + phase
            nxt = local + (_NSLOT - 1)

            @pl.when(nxt < _RH)
            def _():
                fire((phase + _NSLOT - 1) % _NSLOT, nxt)

            consume(phase, local)
        return carry

    for half in range(2):
        hbase = base + half * _RH
        pltpu.sync_copy(idx_hbm.at[pl.ds(hbase, _RH)], idxbuf)
        for p in range(_NSLOT - 1):
            fire(p, p)
        lax.fori_loop(0, _RH // _NSLOT, outer, 0)
        pltpu.sync_copy(outbuf, out_hbm.at[pl.ds(hbase, _RH)])


_sc_pool = functools.partial(
    pl.kernel,
    mesh=plsc.VectorSubcoreMesh(core_axis_name="c", subcore_axis_name="s"),
    compiler_params=pltpu.CompilerParams(
        needs_layout_passes=False, use_tc_tiling_on_sc=False),
    out_type=jax.ShapeDtypeStruct((_NR, _D), jnp.float32),
    scratch_types=[
        pltpu.VMEM((_RH, _LP), jnp.int32),
        pltpu.VMEM((_NSLOT, _LP, _D), jnp.bfloat16),
        pltpu.VMEM((_RH, _D), jnp.float32),
        pltpu.VMEM((_D,), jnp.bfloat16),
    ] + [pltpu.SemaphoreType.DMA] * _NSLOT,
)(_sc_pool_body)


def _ln_body(x_ref, g_ref, b_ref, o_ref):
    x = x_ref[...]
    mu = jnp.mean(x, axis=-1, keepdims=True)
    xc = x - mu
    var = jnp.mean(xc * xc, axis=-1, keepdims=True)
    o_ref[...] = xc * lax.rsqrt(var + _EPS) * g_ref[...] + b_ref[...]


def _layernorm(pooled, gamma, beta):
    blk = 1024
    return pl.pallas_call(
        _ln_body,
        grid=(_NR // blk,),
        in_specs=[
            pl.BlockSpec((blk, _D), lambda i: (i, 0)),
            pl.BlockSpec((1, _D), lambda i: (0, 0)),
            pl.BlockSpec((1, _D), lambda i: (0, 0)),
        ],
        out_specs=pl.BlockSpec((blk, _D), lambda i: (i, 0)),
        out_shape=jax.ShapeDtypeStruct((_NR, _D), jnp.float32),
    )(pooled, gamma, beta)


def kernel(x_s, x_t, table, gamma, beta):
    idx = jnp.concatenate(
        [x_s.astype(jnp.int32), x_t.astype(jnp.int32)], axis=0)
    idx = jnp.pad(idx, ((0, 0), (0, _LP - _L)))
    pooled = _sc_pool(idx, table.astype(jnp.bfloat16))
    pooled = pooled.reshape(_NR, 2, 2, _LANES).transpose(0, 1, 3, 2)
    pooled = pooled.reshape(_NR, _D)
    out = _layernorm(pooled, gamma.reshape(1, _D), beta.reshape(1, _D))
    return out[:_B], out[_B:]
